# Initial kernel scaffold; baseline (speedup 1.0000x reference)
import functools

import jax
import jax.numpy as jnp
from jax import lax
from jax.experimental import pallas as pl
from jax.experimental.pallas import tpu as pltpu
from jax.experimental.pallas import tpu_sc as plsc

N_NODES = 10000
N_PAD = 10240
D = 128
E_EDGES = 320000
NUM_TILES = 32
CHUNK = 128
CHUNKS_PER_TILE = 80
E_PAD = NUM_TILES * CHUNKS_PER_TILE * CHUNK
PAD_SRC = N_PAD - 1
PAD_DST = N_PAD - 2
ROWS_PER_TILE = N_PAD // 16
TC_BLOCK = 1280

_MESH = plsc.VectorSubcoreMesh(
    core_axis_name="c", subcore_axis_name="s", num_cores=2, num_subcores=16
)


def _deg_body(dst_hbm, ones_hbm, zeros_hbm, deg_hbm, dst_v, ones_v, acc, sem):
    c = lax.axis_index("c")
    s = lax.axis_index("s")
    w = c * 16 + s
    pltpu.sync_copy(dst_hbm.at[w], dst_v)
    pltpu.sync_copy(ones_hbm, ones_v)
    pltpu.sync_copy(zeros_hbm.at[pl.ds(s * ROWS_PER_TILE, ROWS_PER_TILE)],
                    acc.at[pl.ds(s * ROWS_PER_TILE, ROWS_PER_TILE)])
    plsc.subcore_barrier()

    def body(j, carry):
        pltpu.sync_copy(ones_v, acc.at[dst_v.at[j]], add=True)
        return carry

    lax.fori_loop(0, CHUNKS_PER_TILE, body, 0)
    plsc.subcore_barrier()
    pltpu.sync_copy(acc.at[pl.ds(s * ROWS_PER_TILE, ROWS_PER_TILE)],
                    deg_hbm.at[c, pl.ds(s * ROWS_PER_TILE, ROWS_PER_TILE)])


_deg_kernel = functools.partial(
    pl.kernel,
    out_type=jax.ShapeDtypeStruct((2, N_PAD, 8), jnp.float32),
    mesh=_MESH,
    scratch_types=[
        pltpu.VMEM((CHUNKS_PER_TILE, CHUNK), jnp.int32),
        pltpu.VMEM((CHUNK, 8), jnp.float32),
        pltpu.VMEM_SHARED((N_PAD, 8), jnp.float32),
        pltpu.SemaphoreType.DMA,
    ],
)(_deg_body)


def _agg_body(src_hbm, dst_hbm, hs_hbm, zeros_hbm, out_hbm,
              src_v, dst_v, buf_a, buf_b, acc, sem_a, sem_b):
    c = lax.axis_index("c")
    s = lax.axis_index("s")
    w = c * 16 + s
    pltpu.sync_copy(src_hbm.at[w], src_v)
    pltpu.sync_copy(dst_hbm.at[w], dst_v)
    pltpu.sync_copy(zeros_hbm.at[pl.ds(s * ROWS_PER_TILE, ROWS_PER_TILE)],
                    acc.at[pl.ds(s * ROWS_PER_TILE, ROWS_PER_TILE)])
    plsc.subcore_barrier()

    def gather(j, buf, sem):
        return pltpu.make_async_copy(hs_hbm.at[src_v.at[j]], buf, sem)

    gather(0, buf_a, sem_a).start()

    def body(i, carry):
        j = 2 * i
        gather(j + 1, buf_b, sem_b).start()
        gather(j, buf_a, sem_a).wait()
        pltpu.sync_copy(buf_a, acc.at[dst_v.at[j]], add=True)
        gather(j + 2, buf_a, sem_a).start()
        gather(j + 1, buf_b, sem_b).wait()
        pltpu.sync_copy(buf_b, acc.at[dst_v.at[j + 1]], add=True)
        return carry

    lax.fori_loop(0, CHUNKS_PER_TILE // 2 - 1, body, 0)
    j = CHUNKS_PER_TILE - 2
    gather(j + 1, buf_b, sem_b).start()
    gather(j, buf_a, sem_a).wait()
    pltpu.sync_copy(buf_a, acc.at[dst_v.at[j]], add=True)
    gather(j + 1, buf_b, sem_b).wait()
    pltpu.sync_copy(buf_b, acc.at[dst_v.at[j + 1]], add=True)

    plsc.subcore_barrier()
    pltpu.sync_copy(acc.at[pl.ds(s * ROWS_PER_TILE, ROWS_PER_TILE)],
                    out_hbm.at[c, pl.ds(s * ROWS_PER_TILE, ROWS_PER_TILE)])


_agg_kernel = functools.partial(
    pl.kernel,
    out_type=jax.ShapeDtypeStruct((2, N_PAD, D), jnp.float32),
    mesh=_MESH,
    scratch_types=[
        pltpu.VMEM((CHUNKS_PER_TILE, CHUNK), jnp.int32),
        pltpu.VMEM((CHUNKS_PER_TILE, CHUNK), jnp.int32),
        pltpu.VMEM((CHUNK, D), jnp.float32),
        pltpu.VMEM((CHUNK, D), jnp.float32),
        pltpu.VMEM_SHARED((N_PAD, D), jnp.float32),
        pltpu.SemaphoreType.DMA,
        pltpu.SemaphoreType.DMA,
    ],
)(_agg_body)


def _dinv(dega_ref, degb_ref):
    deg = dega_ref[:, :1] + degb_ref[:, :1] + 1.0
    return lax.rsqrt(deg)


def _tc1_body(x_ref, wl_ref, bl_ref, dega_ref, degb_ref, o_ref):
    h0 = jnp.dot(x_ref[...], wl_ref[...],
                 preferred_element_type=jnp.float32) + bl_ref[...]
    o_ref[...] = _dinv(dega_ref, degb_ref) * h0


def _tc2_body(acca_ref, accb_ref, hs0_ref, dega_ref, degb_ref,
              w1_ref, b1_ref, o_ref):
    dinv = _dinv(dega_ref, degb_ref)
    a = dinv * (acca_ref[...] + accb_ref[...] + hs0_ref[...])
    h1 = jnp.maximum(
        jnp.dot(a, w1_ref[...], preferred_element_type=jnp.float32)
        + b1_ref[...], 0.0)
    o_ref[...] = dinv * h1


def _tc3_body(acca_ref, accb_ref, hs1_ref, dega_ref, degb_ref,
              w23_ref, b23_ref, o_ref):
    dinv = _dinv(dega_ref, degb_ref)
    a = dinv * (acca_ref[...] + accb_ref[...] + hs1_ref[...])
    f = jnp.dot(a, w23_ref[...], preferred_element_type=jnp.float32) \
        + b23_ref[...]
    col = lax.broadcasted_iota(jnp.int32, f.shape, 1)
    o_ref[...] = jnp.where(col >= 64, jnp.maximum(f, 0.0), f)


def _row_spec(width):
    return pl.BlockSpec((TC_BLOCK, width), lambda i: (i, 0))


def _full_spec(shape):
    return pl.BlockSpec(shape, lambda i: tuple(0 for _ in shape))


def _make_tc(body, n_feature_inputs):
    in_specs = [_row_spec(D)] * n_feature_inputs
    in_specs += [_row_spec(8), _row_spec(8)]
    in_specs += [_full_spec((D, D)), _full_spec((1, D))]
    return pl.pallas_call(
        body,
        grid=(N_PAD // TC_BLOCK,),
        in_specs=in_specs,
        out_specs=_row_spec(D),
        out_shape=jax.ShapeDtypeStruct((N_PAD, D), jnp.float32),
    )


_tc1 = _make_tc(_tc1_body, 1)
_tc2 = _make_tc(_tc2_body, 3)
_tc3 = _make_tc(_tc3_body, 3)


def kernel(x, edge_index, Wl, bl, W1, b1, W2, b2, W3, b3):
    src = edge_index[0]
    dst = edge_index[1]
    pad_e = E_PAD - E_EDGES
    src = jnp.concatenate(
        [src, jnp.full((pad_e,), PAD_SRC, jnp.int32)]).reshape(
            NUM_TILES, CHUNKS_PER_TILE, CHUNK)
    dst = jnp.concatenate(
        [dst, jnp.full((pad_e,), PAD_DST, jnp.int32)]).reshape(
            NUM_TILES, CHUNKS_PER_TILE, CHUNK)

    x_pad = jnp.zeros((N_PAD, D), jnp.float32).at[:N_NODES].set(x)
    zeros_d = jnp.zeros((N_PAD, D), jnp.float32)
    zeros_8 = jnp.zeros((N_PAD, 8), jnp.float32)
    ones_8 = jnp.ones((CHUNK, 8), jnp.float32)

    deg = _deg_kernel(dst, ones_8, zeros_8)
    dega, degb = deg[0], deg[1]

    bl2 = bl.reshape(1, D)
    b12 = b1.reshape(1, D)
    w23 = jnp.concatenate([W2, W3], axis=1)
    b23 = jnp.concatenate([b2, b3]).reshape(1, D)

    hs0 = _tc1(x_pad, Wl, bl2, dega, degb)
    acc1 = _agg_kernel(src, dst, hs0, zeros_d)
    hs1 = _tc2(acc1[0], acc1[1], hs0, dega, degb, W1, b12)
    acc2 = _agg_kernel(src, dst, hs1, zeros_d)
    out = _tc3(acc2[0], acc2[1], hs1, dega, degb, w23, b23)

    mu = out[:N_NODES, :64]
    logvar = out[:N_NODES, 64:]
    return (mu, logvar)


# speedup vs baseline: 9.7993x; 9.7993x over previous
import functools

import jax
import jax.numpy as jnp
from jax import lax
from jax.experimental import pallas as pl
from jax.experimental.pallas import tpu as pltpu
from jax.experimental.pallas import tpu_sc as plsc

N_NODES = 10000
N_PAD = 10240
D = 128
E_EDGES = 320000
NUM_TILES = 32
CHUNK = 128
CHUNKS_PER_TILE = 80
GROUP_CHUNKS = 40
E_PAD = NUM_TILES * CHUNKS_PER_TILE * CHUNK
PAD_SRC = N_PAD - 1
PAD_DST = N_PAD - 2
ROWS_PER_TILE = N_PAD // 16
TC_BLOCK = 1280

_MESH = plsc.VectorSubcoreMesh(
    core_axis_name="c", subcore_axis_name="s", num_cores=2, num_subcores=16
)


def _deg_body(dst_hbm, ones_hbm, zeros_hbm, deg_hbm, dst_v, ones_v, acc, sem):
    c = lax.axis_index("c")
    s = lax.axis_index("s")
    w = c * 16 + s
    pltpu.sync_copy(dst_hbm.at[w], dst_v)
    pltpu.sync_copy(ones_hbm, ones_v)
    pltpu.sync_copy(zeros_hbm.at[pl.ds(s * ROWS_PER_TILE, ROWS_PER_TILE)],
                    acc.at[pl.ds(s * ROWS_PER_TILE, ROWS_PER_TILE)])
    plsc.subcore_barrier()

    def body(j, carry):
        pltpu.sync_copy(ones_v, acc.at[dst_v.at[j]], add=True)
        return carry

    lax.fori_loop(0, CHUNKS_PER_TILE, body, 0)
    plsc.subcore_barrier()
    pltpu.sync_copy(acc.at[pl.ds(s * ROWS_PER_TILE, ROWS_PER_TILE)],
                    deg_hbm.at[c, pl.ds(s * ROWS_PER_TILE, ROWS_PER_TILE)])


_deg_kernel = functools.partial(
    pl.kernel,
    out_type=jax.ShapeDtypeStruct((2, N_PAD, D), jnp.float32),
    mesh=_MESH,
    scratch_types=[
        pltpu.VMEM((CHUNKS_PER_TILE, CHUNK), jnp.int32),
        pltpu.VMEM((CHUNK, D), jnp.float32),
        pltpu.VMEM_SHARED((N_PAD, D), jnp.float32),
        pltpu.SemaphoreType.DMA,
    ],
)(_deg_body)


def _agg_body(src_hbm, dst_hbm, hs_hbm, zeros_hbm, out_hbm,
              src_v, dst_v, buf_a, buf_b, acc, sem_a, sem_b):
    c = lax.axis_index("c")
    s = lax.axis_index("s")
    w = c * 16 + s
    pltpu.sync_copy(zeros_hbm.at[pl.ds(s * ROWS_PER_TILE, ROWS_PER_TILE)],
                    acc.at[pl.ds(s * ROWS_PER_TILE, ROWS_PER_TILE)])
    plsc.subcore_barrier()

    def gather(j, buf, sem):
        return pltpu.make_async_copy(hs_hbm.at[src_v.at[j]], buf, sem)

    for g in range(CHUNKS_PER_TILE // GROUP_CHUNKS):
        pltpu.sync_copy(src_hbm.at[w, pl.ds(g * GROUP_CHUNKS, GROUP_CHUNKS)],
                        src_v)
        pltpu.sync_copy(dst_hbm.at[w, pl.ds(g * GROUP_CHUNKS, GROUP_CHUNKS)],
                        dst_v)
        gather(0, buf_a, sem_a).start()

        def body(i, carry):
            j = 2 * i
            gather(j + 1, buf_b, sem_b).start()
            gather(j, buf_a, sem_a).wait()
            pltpu.sync_copy(buf_a, acc.at[dst_v.at[j]], add=True)
            gather(j + 2, buf_a, sem_a).start()
            gather(j + 1, buf_b, sem_b).wait()
            pltpu.sync_copy(buf_b, acc.at[dst_v.at[j + 1]], add=True)
            return carry

        lax.fori_loop(0, GROUP_CHUNKS // 2 - 1, body, 0)
        j = GROUP_CHUNKS - 2
        gather(j + 1, buf_b, sem_b).start()
        gather(j, buf_a, sem_a).wait()
        pltpu.sync_copy(buf_a, acc.at[dst_v.at[j]], add=True)
        gather(j + 1, buf_b, sem_b).wait()
        pltpu.sync_copy(buf_b, acc.at[dst_v.at[j + 1]], add=True)

    plsc.subcore_barrier()
    pltpu.sync_copy(acc.at[pl.ds(s * ROWS_PER_TILE, ROWS_PER_TILE)],
                    out_hbm.at[c, pl.ds(s * ROWS_PER_TILE, ROWS_PER_TILE)])


_agg_kernel = functools.partial(
    pl.kernel,
    out_type=jax.ShapeDtypeStruct((2, N_PAD, D), jnp.float32),
    mesh=_MESH,
    scratch_types=[
        pltpu.VMEM((GROUP_CHUNKS, CHUNK), jnp.int32),
        pltpu.VMEM((GROUP_CHUNKS, CHUNK), jnp.int32),
        pltpu.VMEM((CHUNK, D), jnp.float32),
        pltpu.VMEM((CHUNK, D), jnp.float32),
        pltpu.VMEM_SHARED((N_PAD, D), jnp.float32),
        pltpu.SemaphoreType.DMA,
        pltpu.SemaphoreType.DMA,
    ],
)(_agg_body)


def _dinv(dega_ref, degb_ref):
    deg = dega_ref[:, :1] + degb_ref[:, :1] + 1.0
    return lax.rsqrt(deg)


def _tc1_body(x_ref, wl_ref, bl_ref, dega_ref, degb_ref, o_ref):
    h0 = jnp.dot(x_ref[...], wl_ref[...],
                 preferred_element_type=jnp.float32) + bl_ref[...]
    o_ref[...] = _dinv(dega_ref, degb_ref) * h0


def _tc2_body(acca_ref, accb_ref, hs0_ref, dega_ref, degb_ref,
              w1_ref, b1_ref, o_ref):
    dinv = _dinv(dega_ref, degb_ref)
    a = dinv * (acca_ref[...] + accb_ref[...] + hs0_ref[...])
    h1 = jnp.maximum(
        jnp.dot(a, w1_ref[...], preferred_element_type=jnp.float32)
        + b1_ref[...], 0.0)
    o_ref[...] = dinv * h1


def _tc3_body(acca_ref, accb_ref, hs1_ref, dega_ref, degb_ref,
              w23_ref, b23_ref, o_ref):
    dinv = _dinv(dega_ref, degb_ref)
    a = dinv * (acca_ref[...] + accb_ref[...] + hs1_ref[...])
    f = jnp.dot(a, w23_ref[...], preferred_element_type=jnp.float32) \
        + b23_ref[...]
    col = lax.broadcasted_iota(jnp.int32, f.shape, 1)
    o_ref[...] = jnp.where(col >= 64, jnp.maximum(f, 0.0), f)


def _row_spec(width):
    return pl.BlockSpec((TC_BLOCK, width), lambda i: (i, 0))


def _full_spec(shape):
    return pl.BlockSpec(shape, lambda i: tuple(0 for _ in shape))


def _make_tc(body, in_specs):
    return pl.pallas_call(
        body,
        grid=(N_PAD // TC_BLOCK,),
        in_specs=in_specs,
        out_specs=_row_spec(D),
        out_shape=jax.ShapeDtypeStruct((N_PAD, D), jnp.float32),
    )


_DEG_SPECS = [_row_spec(D), _row_spec(D)]
_W_SPECS = [_full_spec((D, D)), _full_spec((1, D))]
_tc1 = _make_tc(_tc1_body, [_row_spec(D)] + _W_SPECS + _DEG_SPECS)
_tc2 = _make_tc(_tc2_body, [_row_spec(D)] * 3 + _DEG_SPECS + _W_SPECS)
_tc3 = _make_tc(_tc3_body, [_row_spec(D)] * 3 + _DEG_SPECS + _W_SPECS)


def kernel(x, edge_index, Wl, bl, W1, b1, W2, b2, W3, b3):
    src = edge_index[0]
    dst = edge_index[1]
    pad_e = E_PAD - E_EDGES
    src = jnp.concatenate(
        [src, jnp.full((pad_e,), PAD_SRC, jnp.int32)]).reshape(
            NUM_TILES, CHUNKS_PER_TILE, CHUNK)
    dst = jnp.concatenate(
        [dst, jnp.full((pad_e,), PAD_DST, jnp.int32)]).reshape(
            NUM_TILES, CHUNKS_PER_TILE, CHUNK)

    x_pad = jnp.zeros((N_PAD, D), jnp.float32).at[:N_NODES].set(x)
    zeros_d = jnp.zeros((N_PAD, D), jnp.float32)
    ones_row = jnp.ones((CHUNK, D), jnp.float32)

    deg = _deg_kernel(dst, ones_row, zeros_d)
    dega, degb = deg[0], deg[1]

    bl2 = bl.reshape(1, D)
    b12 = b1.reshape(1, D)
    w23 = jnp.concatenate([W2, W3], axis=1)
    b23 = jnp.concatenate([b2, b3]).reshape(1, D)

    hs0 = _tc1(x_pad, Wl, bl2, dega, degb)
    acc1 = _agg_kernel(src, dst, hs0, zeros_d)
    hs1 = _tc2(acc1[0], acc1[1], hs0, dega, degb, W1, b12)
    acc2 = _agg_kernel(src, dst, hs1, zeros_d)
    out = _tc3(acc2[0], acc2[1], hs1, dega, degb, w23, b23)

    mu = out[:N_NODES, :64]
    logvar = out[:N_NODES, 64:]
    return (mu, logvar)


# speedup vs baseline: 29.7551x; 3.0364x over previous
import functools

import jax
import jax.numpy as jnp
from jax import lax
from jax.experimental import pallas as pl
from jax.experimental.pallas import tpu as pltpu
from jax.experimental.pallas import tpu_sc as plsc

N_NODES = 10000
N_PAD = 10240
D = 128
E_EDGES = 320000
NUM_TILES = 32
CHUNK = 128
CHUNKS_PER_TILE = 80
GROUP_CHUNKS = 40
E_PAD = NUM_TILES * CHUNKS_PER_TILE * CHUNK
PAD_SRC = N_PAD - 1
PAD_DST = N_PAD - 2
ROWS_PER_TILE = N_PAD // 16
TC_BLOCK = 1280

_MESH = plsc.VectorSubcoreMesh(
    core_axis_name="c", subcore_axis_name="s", num_cores=2, num_subcores=16
)


def _deg_body(dst_hbm, ones_hbm, zeros_hbm, deg_hbm, dst_v, ones_v, acc, sem):
    c = lax.axis_index("c")
    s = lax.axis_index("s")
    w = c * 16 + s
    pltpu.sync_copy(dst_hbm.at[w], dst_v)
    pltpu.sync_copy(ones_hbm, ones_v)
    pltpu.sync_copy(zeros_hbm.at[pl.ds(s * ROWS_PER_TILE, ROWS_PER_TILE)],
                    acc.at[pl.ds(s * ROWS_PER_TILE, ROWS_PER_TILE)])
    plsc.subcore_barrier()

    def body(j, carry):
        pltpu.sync_copy(ones_v, acc.at[dst_v.at[j]], add=True)
        return carry

    lax.fori_loop(0, CHUNKS_PER_TILE, body, 0)
    plsc.subcore_barrier()
    pltpu.sync_copy(acc.at[pl.ds(s * ROWS_PER_TILE, ROWS_PER_TILE)],
                    deg_hbm.at[c, pl.ds(s * ROWS_PER_TILE, ROWS_PER_TILE)])


_deg_kernel = functools.partial(
    pl.kernel,
    out_type=jax.ShapeDtypeStruct((2, N_PAD, D), jnp.float32),
    mesh=_MESH,
    scratch_types=[
        pltpu.VMEM((CHUNKS_PER_TILE, CHUNK), jnp.int32),
        pltpu.VMEM((CHUNK, D), jnp.float32),
        pltpu.VMEM_SHARED((N_PAD, D), jnp.float32),
        pltpu.SemaphoreType.DMA,
    ],
)(_deg_body)


def _agg_body(src_hbm, dst_hbm, hs_hbm, zeros_hbm, out_hbm,
              src_v, dst_v, buf_a, buf_b, acc, sem_a, sem_b):
    c = lax.axis_index("c")
    s = lax.axis_index("s")
    w = c * 16 + s
    pltpu.sync_copy(zeros_hbm.at[pl.ds(s * ROWS_PER_TILE, ROWS_PER_TILE)],
                    acc.at[pl.ds(s * ROWS_PER_TILE, ROWS_PER_TILE)])
    plsc.subcore_barrier()

    def gather(j, buf, sem):
        return pltpu.make_async_copy(hs_hbm.at[src_v.at[j]], buf, sem)

    for g in range(CHUNKS_PER_TILE // GROUP_CHUNKS):
        pltpu.sync_copy(src_hbm.at[w, pl.ds(g * GROUP_CHUNKS, GROUP_CHUNKS)],
                        src_v)
        pltpu.sync_copy(dst_hbm.at[w, pl.ds(g * GROUP_CHUNKS, GROUP_CHUNKS)],
                        dst_v)
        gather(0, buf_a, sem_a).start()

        def body(i, carry):
            j = 2 * i
            gather(j + 1, buf_b, sem_b).start()
            gather(j, buf_a, sem_a).wait()
            pltpu.sync_copy(buf_a, acc.at[dst_v.at[j]], add=True)
            gather(j + 2, buf_a, sem_a).start()
            gather(j + 1, buf_b, sem_b).wait()
            pltpu.sync_copy(buf_b, acc.at[dst_v.at[j + 1]], add=True)
            return carry

        lax.fori_loop(0, GROUP_CHUNKS // 2 - 1, body, 0)
        j = GROUP_CHUNKS - 2
        gather(j + 1, buf_b, sem_b).start()
        gather(j, buf_a, sem_a).wait()
        pltpu.sync_copy(buf_a, acc.at[dst_v.at[j]], add=True)
        gather(j + 1, buf_b, sem_b).wait()
        pltpu.sync_copy(buf_b, acc.at[dst_v.at[j + 1]], add=True)

    plsc.subcore_barrier()
    pltpu.sync_copy(acc.at[pl.ds(s * ROWS_PER_TILE, ROWS_PER_TILE)],
                    out_hbm.at[c, pl.ds(s * ROWS_PER_TILE, ROWS_PER_TILE)])


_agg_kernel = functools.partial(
    pl.kernel,
    out_type=jax.ShapeDtypeStruct((2, N_PAD, D), jnp.float32),
    mesh=_MESH,
    scratch_types=[
        pltpu.VMEM((GROUP_CHUNKS, CHUNK), jnp.int32),
        pltpu.VMEM((GROUP_CHUNKS, CHUNK), jnp.int32),
        pltpu.VMEM((CHUNK, D), jnp.float32),
        pltpu.VMEM((CHUNK, D), jnp.float32),
        pltpu.VMEM_SHARED((N_PAD, D), jnp.float32),
        pltpu.SemaphoreType.DMA,
        pltpu.SemaphoreType.DMA,
    ],
)(_agg_body)


def _dinv(dega_ref, degb_ref):
    deg = dega_ref[:, :1] + degb_ref[:, :1] + 1.0
    return lax.rsqrt(deg)


def _tc1_body(x_ref, wl_ref, bl_ref, dega_ref, degb_ref, o_ref):
    h0 = jnp.dot(x_ref[...], wl_ref[...],
                 preferred_element_type=jnp.float32) + bl_ref[...]
    o_ref[...] = _dinv(dega_ref, degb_ref) * h0


def _tc2_body(acca_ref, accb_ref, hs0_ref, dega_ref, degb_ref,
              w1_ref, b1_ref, o_ref):
    dinv = _dinv(dega_ref, degb_ref)
    a = dinv * (acca_ref[...] + accb_ref[...] + hs0_ref[...])
    h1 = jnp.maximum(
        jnp.dot(a, w1_ref[...], preferred_element_type=jnp.float32)
        + b1_ref[...], 0.0)
    o_ref[...] = dinv * h1


def _tc3_body(acca_ref, accb_ref, hs1_ref, dega_ref, degb_ref,
              w23_ref, b23_ref, o_ref):
    dinv = _dinv(dega_ref, degb_ref)
    a = dinv * (acca_ref[...] + accb_ref[...] + hs1_ref[...])
    f = jnp.dot(a, w23_ref[...], preferred_element_type=jnp.float32) \
        + b23_ref[...]
    col = lax.broadcasted_iota(jnp.int32, f.shape, 1)
    o_ref[...] = jnp.where(col >= 64, jnp.maximum(f, 0.0), f)


def _row_spec(width):
    return pl.BlockSpec((TC_BLOCK, width), lambda i: (i, 0))


def _full_spec(shape):
    return pl.BlockSpec(shape, lambda i: tuple(0 for _ in shape))


def _make_tc(body, in_specs):
    return pl.pallas_call(
        body,
        grid=(N_PAD // TC_BLOCK,),
        in_specs=in_specs,
        out_specs=_row_spec(D),
        out_shape=jax.ShapeDtypeStruct((N_PAD, D), jnp.float32),
    )


_DEG_SPECS = [_row_spec(D), _row_spec(D)]
_W_SPECS = [_full_spec((D, D)), _full_spec((1, D))]
_tc1 = _make_tc(_tc1_body, [_row_spec(D)] + _W_SPECS + _DEG_SPECS)
_tc2 = _make_tc(_tc2_body, [_row_spec(D)] * 3 + _DEG_SPECS + _W_SPECS)
_tc3 = _make_tc(_tc3_body, [_row_spec(D)] * 3 + _DEG_SPECS + _W_SPECS)


def kernel(x, edge_index, Wl, bl, W1, b1, W2, b2, W3, b3):
    src = edge_index[0]
    dst = edge_index[1]
    pad_e = E_PAD - E_EDGES
    pad_idx = N_NODES + (jnp.arange(pad_e, dtype=jnp.int32) % (N_PAD - N_NODES))
    src = jnp.concatenate([src, pad_idx]).reshape(
        NUM_TILES, CHUNKS_PER_TILE, CHUNK)
    dst = jnp.concatenate([dst, pad_idx]).reshape(
        NUM_TILES, CHUNKS_PER_TILE, CHUNK)

    x_pad = jnp.zeros((N_PAD, D), jnp.float32).at[:N_NODES].set(x)
    zeros_d = jnp.zeros((N_PAD, D), jnp.float32)
    ones_row = jnp.ones((CHUNK, D), jnp.float32)

    deg = _deg_kernel(dst, ones_row, zeros_d)
    dega, degb = deg[0], deg[1]

    bl2 = bl.reshape(1, D)
    b12 = b1.reshape(1, D)
    w23 = jnp.concatenate([W2, W3], axis=1)
    b23 = jnp.concatenate([b2, b3]).reshape(1, D)

    hs0 = _tc1(x_pad, Wl, bl2, dega, degb)
    acc1 = _agg_kernel(src, dst, hs0, zeros_d)
    hs1 = _tc2(acc1[0], acc1[1], hs0, dega, degb, W1, b12)
    acc2 = _agg_kernel(src, dst, hs1, zeros_d)
    out = _tc3(acc2[0], acc2[1], hs1, dega, degb, w23, b23)

    mu = out[:N_NODES, :64]
    logvar = out[:N_NODES, 64:]
    return (mu, logvar)
